# baseline (speedup 1.0000x reference)
import jax
import jax.numpy as jnp
from jax.experimental import pallas as pl
from jax.experimental.pallas import tpu as pltpu

N = 10000
D = 256
BM = 400
NB = N // BM

_P = jax.lax.Precision.DEFAULT


def _fused_kernel(a_ref, x_hbm, w0_ref, b0_ref, w1_ref, b1_ref,
                  out_ref, x_ref, s1_ref, sem):
    i = pl.program_id(0)

    @pl.when(i == 0)
    def _stage0():
        copy = pltpu.make_async_copy(x_hbm, x_ref, sem)
        copy.start()
        copy.wait()

    @pl.when(jnp.logical_and(i >= 1, i <= NB))
    def _pass1():
        ib = i - 1
        t = jnp.dot(a_ref[...], x_ref[...],
                    preferred_element_type=jnp.float32, precision=_P)
        h = jnp.maximum(
            jnp.dot(t, w0_ref[...],
                    preferred_element_type=jnp.float32, precision=_P)
            + b0_ref[...],
            0.0,
        )
        s1_ref[pl.ds(ib * BM, BM), :] = jnp.dot(
            h, w1_ref[...], preferred_element_type=jnp.float32, precision=_P)

    @pl.when(i >= NB + 1)
    def _pass2():
        out_ref[...] = jnp.dot(
            a_ref[...], s1_ref[...],
            preferred_element_type=jnp.float32, precision=_P) + b1_ref[...]


def kernel(features, adjacency, W0, b0, W1, b1):
    return pl.pallas_call(
        _fused_kernel,
        grid=(2 * NB + 1,),
        in_specs=[
            pl.BlockSpec(
                (BM, N),
                lambda i: (jnp.where(i <= NB, jnp.maximum(i - 1, 0), 2 * NB - i), 0),
            ),
            pl.BlockSpec(memory_space=pltpu.MemorySpace.HBM),
            pl.BlockSpec((D, D), lambda i: (0, 0)),
            pl.BlockSpec((1, D), lambda i: (0, 0)),
            pl.BlockSpec((D, D), lambda i: (0, 0)),
            pl.BlockSpec((1, D), lambda i: (0, 0)),
        ],
        out_specs=pl.BlockSpec(
            (BM, D),
            lambda i: (jnp.where(i > NB, 2 * NB - i, NB - 1), 0),
        ),
        out_shape=jax.ShapeDtypeStruct((N, D), jnp.float32),
        scratch_shapes=[
            pltpu.VMEM((N, D), jnp.float32),
            pltpu.VMEM((N, D), jnp.float32),
            pltpu.SemaphoreType.DMA,
        ],
    )(
        adjacency,
        features,
        W0,
        b0.reshape(1, D),
        W1,
        b1.reshape(1, D),
    )


# speedup vs baseline: 1.0048x; 1.0048x over previous
import jax
import jax.numpy as jnp
from jax.experimental import pallas as pl
from jax.experimental.pallas import tpu as pltpu

N = 10000
D = 256
BM = 200
NB = N // BM
NBUF = 4

_P = jax.lax.Precision.DEFAULT


def _a_copy(a_hbm, bufs, sems, blk, slot):
    return pltpu.make_async_copy(
        a_hbm.at[pl.ds(blk * BM, BM), :],
        bufs.at[slot],
        sems.at[slot],
    )


def _start_blk(a_hbm, bufs, sems, blk):
    slot = jax.lax.rem(blk, NBUF)
    for j in range(NBUF):
        @pl.when(slot == j)
        def _(j=j):
            _a_copy(a_hbm, bufs, sems, blk, j).start()


def _wait_blk(a_hbm, bufs, sems, blk):
    slot = jax.lax.rem(blk, NBUF)
    for j in range(NBUF):
        @pl.when(slot == j)
        def _(j=j):
            _a_copy(a_hbm, bufs, sems, blk, j).wait()


def _fused_kernel(a_hbm, x_hbm, w0_ref, b0_ref, w1_ref, b1_ref,
                  out_ref, x_ref, s1_ref, bufs, t_ref, xsem, sems):
    g = pl.program_id(0)

    @pl.when(g == 0)
    def _stage0():
        xcopy = pltpu.make_async_copy(x_hbm, x_ref, xsem)
        xcopy.start()
        for j in range(NBUF):
            _a_copy(a_hbm, bufs, sems, j, j).start()
        xcopy.wait()

    @pl.when(jnp.logical_and(g >= 1, g <= NB))
    def _pass1():
        b = g - 1
        _wait_blk(a_hbm, bufs, sems, b)
        slot = jax.lax.rem(b, NBUF)
        for j in range(NBUF):
            @pl.when(slot == j)
            def _(j=j):
                t_ref[...] = jnp.dot(bufs[j], x_ref[...],
                                     preferred_element_type=jnp.float32,
                                     precision=_P)
        h = jnp.maximum(
            jnp.dot(t_ref[...], w0_ref[...],
                    preferred_element_type=jnp.float32, precision=_P)
            + b0_ref[...],
            0.0,
        )
        s1_ref[pl.ds(b * BM, BM), :] = jnp.dot(
            h, w1_ref[...], preferred_element_type=jnp.float32, precision=_P)
        nb = b + NBUF

        @pl.when(nb <= NB - 1)
        def _issue():
            _start_blk(a_hbm, bufs, sems, nb)

    @pl.when(g >= NB + 1)
    def _pass2():
        t = g - 1
        b = 2 * NB - 1 - t

        @pl.when(t >= NB + NBUF)
        def _wait():
            _wait_blk(a_hbm, bufs, sems, b)

        slot = jax.lax.rem(b, NBUF)
        for j in range(NBUF):
            @pl.when(slot == j)
            def _(j=j):
                out_ref[...] = jnp.dot(bufs[j], s1_ref[...],
                                       preferred_element_type=jnp.float32,
                                       precision=_P) + b1_ref[...]
        nb2 = 2 * NB - 5 - t

        @pl.when(nb2 >= 0)
        def _issue2():
            _start_blk(a_hbm, bufs, sems, nb2)


def kernel(features, adjacency, W0, b0, W1, b1):
    return pl.pallas_call(
        _fused_kernel,
        grid=(2 * NB + 1,),
        in_specs=[
            pl.BlockSpec(memory_space=pltpu.MemorySpace.HBM),
            pl.BlockSpec(memory_space=pltpu.MemorySpace.HBM),
            pl.BlockSpec((D, D), lambda g: (0, 0)),
            pl.BlockSpec((1, D), lambda g: (0, 0)),
            pl.BlockSpec((D, D), lambda g: (0, 0)),
            pl.BlockSpec((1, D), lambda g: (0, 0)),
        ],
        out_specs=pl.BlockSpec(
            (BM, D),
            lambda g: (jnp.where(g >= NB + 1, 2 * NB - g, NB - 1), 0),
        ),
        out_shape=jax.ShapeDtypeStruct((N, D), jnp.float32),
        scratch_shapes=[
            pltpu.VMEM((N, D), jnp.float32),
            pltpu.VMEM((N, D), jnp.float32),
            pltpu.VMEM((NBUF, BM, N), jnp.float32),
            pltpu.VMEM((BM, D), jnp.float32),
            pltpu.SemaphoreType.DMA,
            pltpu.SemaphoreType.DMA((NBUF,)),
        ],
    )(
        adjacency,
        features,
        W0,
        b0.reshape(1, D),
        W1,
        b1.reshape(1, D),
    )
